# baseline (speedup 1.0000x reference)
import functools

import jax
import jax.numpy as jnp
from jax import lax
from jax.experimental import pallas as pl
from jax.experimental.pallas import tpu as pltpu
from jax.experimental.pallas import tpu_sc as plsc

D_MODEL = 64
N_CODES = 8192
N_ROWS = 8192
ROW_BLK = 512
BETA = 0.25
_WIN = 4096


def _argmin_body(z_ref, emb_ref, z2_ref, e2_ref, idx_ref, loss_ref):
    z = z_ref[...]
    zn = z * (-2.0)
    z2 = z2_ref[...][:, None]

    av = jnp.full((ROW_BLK,), jnp.inf, jnp.float32)
    tv = jnp.zeros((ROW_BLK,), jnp.float32)
    ai = jnp.zeros((ROW_BLK,), jnp.int32)
    for j in range(N_CODES // _WIN):
        e = emb_ref[j * _WIN:(j + 1) * _WIN, :]
        e2 = e2_ref[j * _WIN:(j + 1) * _WIN]
        mmn = lax.dot_general(zn, e, (((1,), (1,)), ((), ())),
                              preferred_element_type=jnp.float32)
        dw = (z2 + e2[None, :]) + mmn
        wv = jnp.min(dw, axis=1)
        iota = lax.broadcasted_iota(jnp.int32, dw.shape, 1) + j * _WIN
        wi = jnp.min(jnp.where(dw == wv[:, None], iota, N_CODES), axis=1)
        lt = av < wv
        keep = lt | ((av == wv) & (ai < wi))
        ai = jnp.where(keep, ai, wi)
        tv = jnp.where(keep, tv, wv)
        av = jnp.where(lt, av, wv).astype(jnp.bfloat16).astype(jnp.float32)
    idx_ref[...] = ai

    i = pl.program_id(0)

    @pl.when(i == 0)
    def _():
        loss_ref[0, 0] = 0.0

    loss_ref[0, 0] += jnp.sum(tv)

    @pl.when(i == pl.num_programs(0) - 1)
    def _():
        loss_ref[0, 0] = loss_ref[0, 0] * ((1.0 + BETA) / (N_ROWS * D_MODEL))


def _distance_argmin(zf, embedding, z2, e2):
    return pl.pallas_call(
        _argmin_body,
        grid=(N_ROWS // ROW_BLK,),
        in_specs=[
            pl.BlockSpec((ROW_BLK, D_MODEL), lambda i: (i, 0)),
            pl.BlockSpec((N_CODES, D_MODEL), lambda i: (0, 0)),
            pl.BlockSpec((ROW_BLK,), lambda i: (i,)),
            pl.BlockSpec((N_CODES,), lambda i: (0,)),
        ],
        out_specs=[
            pl.BlockSpec((ROW_BLK,), lambda i: (i,)),
            pl.BlockSpec(memory_space=pltpu.SMEM,
                         block_shape=(1, 1), index_map=lambda i: (0, 0)),
        ],
        out_shape=[
            jax.ShapeDtypeStruct((N_ROWS,), jnp.int32),
            jax.ShapeDtypeStruct((1, 1), jnp.float32),
        ],
    )(zf, embedding, z2, e2)


_NW = 32
_B_PER_W = N_ROWS // _NW
_CHUNK = 128
_N_CHUNK = _B_PER_W // _CHUNK
_D_PAD = 128


@functools.cache
def _sc_gather_build():
    @functools.partial(
        pl.kernel,
        mesh=plsc.VectorSubcoreMesh(core_axis_name="c", subcore_axis_name="s"),
        out_type=jax.ShapeDtypeStruct((N_ROWS, _D_PAD), jnp.float32),
        scratch_types=[
            pltpu.VMEM((_N_CHUNK, _CHUNK), jnp.int32),
            pltpu.VMEM((_CHUNK, _D_PAD), jnp.float32),
            pltpu.SemaphoreType.DMA,
        ],
    )
    def _sc_gather(table_hbm, idx_hbm, out_hbm, idx_v, rows_v, sem):
        wid = lax.axis_index("s") * 2 + lax.axis_index("c")
        base = wid * _B_PER_W
        pltpu.sync_copy(idx_hbm.at[pl.ds(wid * _N_CHUNK, _N_CHUNK)], idx_v)
        for j in range(_N_CHUNK):
            pltpu.async_copy(table_hbm.at[idx_v.at[j]], rows_v, sem).wait()
            pltpu.sync_copy(rows_v,
                            out_hbm.at[pl.ds(base + j * _CHUNK, _CHUNK)])

    return _sc_gather


def kernel(z, embedding):
    B, P, d = z.shape
    zf = z.reshape(-1, d)
    z2 = jnp.sum(zf ** 2, axis=1, keepdims=True).reshape(-1)
    e2 = jnp.sum(embedding ** 2, axis=1)
    idx, loss = _distance_argmin(zf, embedding, z2, e2)
    table = jnp.pad(embedding, ((0, 0), (0, _D_PAD - D_MODEL)))
    z_q = _sc_gather_build()(table, idx.reshape(_NW * _N_CHUNK, _CHUNK))
    return (z_q[:, :D_MODEL], loss.reshape(()), idx.reshape(N_ROWS, 1))


# speedup vs baseline: 1.0876x; 1.0876x over previous
import functools

import jax
import jax.numpy as jnp
from jax import lax
from jax.experimental import pallas as pl
from jax.experimental.pallas import tpu as pltpu
from jax.experimental.pallas import tpu_sc as plsc

D_MODEL = 64
N_CODES = 8192
N_ROWS = 8192
ROW_BLK = 512
BETA = 0.25
_WIN = 4096


def _argmin_body(z_ref, emb_ref, z2_ref, e2_ref, idx_ref, loss_ref):
    z = z_ref[...]
    z2 = z2_ref[...][:, None]

    av = jnp.full((ROW_BLK,), jnp.inf, jnp.float32)
    tv = jnp.zeros((ROW_BLK,), jnp.float32)
    ai = jnp.zeros((ROW_BLK,), jnp.int32)
    for j in range(N_CODES // _WIN):
        e = emb_ref[j * _WIN:(j + 1) * _WIN, :]
        e2 = e2_ref[j * _WIN:(j + 1) * _WIN]
        mm = lax.dot_general(z, e, (((1,), (1,)), ((), ())),
                             preferred_element_type=jnp.float32)
        dw = (z2 + e2[None, :]) - 2.0 * mm
        wv = jnp.min(dw, axis=1)
        wi = jnp.argmin(dw, axis=1).astype(jnp.int32) + j * _WIN
        lt = av < wv
        keep = lt | ((av == wv) & (ai < wi))
        ai = jnp.where(keep, ai, wi)
        tv = jnp.where(keep, tv, wv)
        av = jnp.where(lt, av, wv).astype(jnp.bfloat16).astype(jnp.float32)
    idx_ref[...] = ai

    i = pl.program_id(0)

    @pl.when(i == 0)
    def _():
        loss_ref[0, 0] = 0.0

    loss_ref[0, 0] += jnp.sum(tv)

    @pl.when(i == pl.num_programs(0) - 1)
    def _():
        loss_ref[0, 0] = loss_ref[0, 0] * ((1.0 + BETA) / (N_ROWS * D_MODEL))


def _distance_argmin(zf, embedding, z2, e2):
    return pl.pallas_call(
        _argmin_body,
        grid=(N_ROWS // ROW_BLK,),
        in_specs=[
            pl.BlockSpec((ROW_BLK, D_MODEL), lambda i: (i, 0)),
            pl.BlockSpec((N_CODES, D_MODEL), lambda i: (0, 0)),
            pl.BlockSpec((ROW_BLK,), lambda i: (i,)),
            pl.BlockSpec((N_CODES,), lambda i: (0,)),
        ],
        out_specs=[
            pl.BlockSpec((ROW_BLK,), lambda i: (i,)),
            pl.BlockSpec(memory_space=pltpu.SMEM,
                         block_shape=(1, 1), index_map=lambda i: (0, 0)),
        ],
        out_shape=[
            jax.ShapeDtypeStruct((N_ROWS,), jnp.int32),
            jax.ShapeDtypeStruct((1, 1), jnp.float32),
        ],
    )(zf, embedding, z2, e2)


_NW = 32
_B_PER_W = N_ROWS // _NW
_CHUNK = 128
_N_CHUNK = _B_PER_W // _CHUNK
_D_PAD = 128


@functools.cache
def _sc_gather_build():
    @functools.partial(
        pl.kernel,
        mesh=plsc.VectorSubcoreMesh(core_axis_name="c", subcore_axis_name="s"),
        out_type=jax.ShapeDtypeStruct((N_ROWS, _D_PAD), jnp.float32),
        scratch_types=[
            pltpu.VMEM((_N_CHUNK, _CHUNK), jnp.int32),
            pltpu.VMEM((_CHUNK, _D_PAD), jnp.float32),
            pltpu.SemaphoreType.DMA,
        ],
    )
    def _sc_gather(table_hbm, idx_hbm, out_hbm, idx_v, rows_v, sem):
        wid = lax.axis_index("s") * 2 + lax.axis_index("c")
        base = wid * _B_PER_W
        pltpu.sync_copy(idx_hbm.at[pl.ds(wid * _N_CHUNK, _N_CHUNK)], idx_v)
        for j in range(_N_CHUNK):
            pltpu.async_copy(table_hbm.at[idx_v.at[j]], rows_v, sem).wait()
            pltpu.sync_copy(rows_v,
                            out_hbm.at[pl.ds(base + j * _CHUNK, _CHUNK)])

    return _sc_gather


def kernel(z, embedding):
    B, P, d = z.shape
    zf = z.reshape(-1, d)
    z2 = jnp.sum(zf ** 2, axis=1, keepdims=True).reshape(-1)
    e2 = jnp.sum(embedding ** 2, axis=1)
    idx, loss = _distance_argmin(zf, embedding, z2, e2)
    table = jnp.pad(embedding, ((0, 0), (0, _D_PAD - D_MODEL)))
    z_q = _sc_gather_build()(table, idx.reshape(_NW * _N_CHUNK, _CHUNK))
    return (z_q[:, :D_MODEL], loss.reshape(()), idx.reshape(N_ROWS, 1))
